# baseline (speedup 1.0000x reference)
import functools

import jax
import jax.numpy as jnp
from jax import lax
from jax.experimental import pallas as pl
from jax.experimental.pallas import tpu as pltpu
from jax.experimental.pallas import tpu_sc as plsc

B, S, D, F, E = 2, 2048, 1024, 4096, 8
CAP = 512
BS = B * S
LANES = 128
SLOTS = E * B * CAP
TRASH = SLOTS
XROWS = SLOTS + 1024
SCHUNK = 512
NC, NS = 2, 16
NW = NC * NS
TPW = BS // NW
CHUNK = 64
NCH = TPW // CHUNK
FB = 1024
NF = F // FB
RSUB = 256


def _router_body(h_ref, wc_ref, logits_ref, hs_ref, dest_ref, destc_ref,
                 counts_ref):
    h = h_ref[...]
    logits = jnp.dot(h, wc_ref[...], preferred_element_type=jnp.float32)
    logits_ref[...] = logits
    lane = lax.broadcasted_iota(jnp.int32, (BS, LANES), 1)
    ml = jnp.where(lane < E, logits, -jnp.inf)
    m = jnp.max(ml, axis=1, keepdims=True)
    eidx = jnp.min(jnp.where(ml == m, lane, LANES), axis=1, keepdims=True)
    pmax = 1.0 / jnp.sum(jnp.exp(ml - m), axis=1, keepdims=True)
    oh = (lane == eidx).astype(jnp.float32)
    ri = lax.broadcasted_iota(jnp.int32, (SCHUNK, SCHUNK), 0)
    ci = lax.broadcasted_iota(jnp.int32, (SCHUNK, SCHUNK), 1)
    tri = (ri >= ci).astype(jnp.float32)
    counts_ref[...] = jnp.zeros((8, LANES), jnp.int32)
    for b in range(B):
        carry = jnp.zeros((1, LANES), jnp.float32)
        for c in range(S // SCHUNK):
            r0 = b * S + c * SCHUNK
            seg = oh[r0:r0 + SCHUNK, :]
            p = jnp.dot(tri, seg, preferred_element_type=jnp.float32) + carry
            carry = carry + jnp.sum(seg, axis=0, keepdims=True)
            prio = jnp.sum(p * seg, axis=1, keepdims=True)
            kept = prio <= CAP
            slot = prio.astype(jnp.int32) - 1
            es = eidx[r0:r0 + SCHUNK, :]
            dst = es * (B * CAP) + b * CAP + slot
            dest_ref[r0:r0 + SCHUNK, :] = jnp.where(kept, dst, TRASH)
            hscale = jnp.where(kept, pmax[r0:r0 + SCHUNK, :], 0.0)
            hs_ref[r0:r0 + SCHUNK, :] = h[r0:r0 + SCHUNK, :] * hscale
        cnt = jnp.minimum(carry, float(CAP)).astype(jnp.int32)
        counts_ref[b:b + 1, :] = cnt
        zlane = jnp.min(jnp.where((carry < CAP) & (lane[:1, :] < E),
                                  lane[:1, :], LANES), axis=1, keepdims=True)
        zcnt = jnp.sum(jnp.where(lane[:1, :] == zlane, carry, 0.0),
                       axis=1, keepdims=True).astype(jnp.int32)
        zrow = zlane * (B * CAP) + b * CAP + zcnt
        for c in range(S // SCHUNK):
            r0 = b * S + c * SCHUNK
            dv = dest_ref[r0:r0 + SCHUNK, :]
            destc_ref[r0:r0 + SCHUNK, :] = jnp.where(dv == TRASH, zrow, dv)


def _router(h2, wcp):
    return pl.pallas_call(
        _router_body,
        out_shape=[
            jax.ShapeDtypeStruct((BS, LANES), jnp.float32),
            jax.ShapeDtypeStruct((BS, D), jnp.float32),
            jax.ShapeDtypeStruct((BS, 1), jnp.int32),
            jax.ShapeDtypeStruct((BS, 1), jnp.int32),
            jax.ShapeDtypeStruct((8, LANES), jnp.int32),
        ],
    )(h2, wcp)


def _sc_dispatch(h2, dest):
    mesh = plsc.VectorSubcoreMesh(core_axis_name="c", subcore_axis_name="s")

    @functools.partial(
        pl.kernel, mesh=mesh,
        out_type=jax.ShapeDtypeStruct((XROWS, D), jnp.float32),
        scratch_types=[
            pltpu.VMEM((NCH, CHUNK), jnp.int32),
            pltpu.VMEM((CHUNK, D), jnp.float32),
            pltpu.SemaphoreType.DMA,
        ],
    )
    def disp(h_hbm, dest_hbm, x_hbm, idx_v, rows_v, sem):
        wid = lax.axis_index("s") * NC + lax.axis_index("c")
        base = wid * TPW
        for j in range(NCH):
            off = base + j * CHUNK
            pltpu.sync_copy(dest_hbm.at[pl.ds(off, CHUNK)], idx_v.at[j])
            pltpu.sync_copy(h_hbm.at[pl.ds(off, CHUNK), :], rows_v)
            pltpu.async_copy(rows_v, x_hbm.at[idx_v.at[j]], sem).wait()

    return disp(h2, dest)


def _ffn_subblock(x_ref, wib, wob, acc_ref, y_ref, f, rows, s0, cntb):
    slot = s0 + lax.broadcasted_iota(jnp.int32, (RSUB, 1), 0)
    xm = jnp.where(slot < cntb, x_ref[rows, :], 0.0)
    hmid = jnp.maximum(
        jnp.dot(xm, wib, preferred_element_type=jnp.float32), 0.0)
    out = jnp.dot(hmid, wob, preferred_element_type=jnp.float32)

    @pl.when(f == 0)
    def _():
        acc_ref[rows, :] = out

    @pl.when((f > 0) & (f < NF - 1))
    def _():
        acc_ref[rows, :] = acc_ref[rows, :] + out

    @pl.when(f == NF - 1)
    def _():
        y_ref[rows, :] = acc_ref[rows, :] + out


def _ffn_body(counts_ref, x_ref, wi_ref, wo_ref, y_ref, acc_ref):
    e = pl.program_id(0)
    f = pl.program_id(1)
    cnt = [counts_ref[0, e], counts_ref[1, e]]
    wib = wi_ref[0]
    wob = wo_ref[0]
    for sb in range(B * CAP // RSUB):
        b, s0 = (sb * RSUB) // CAP, (sb * RSUB) % CAP
        rows = pl.ds(sb * RSUB, RSUB)
        cntb = cnt[b]
        active = cntb > s0

        @pl.when(active)
        def _(rows=rows, s0=s0, cntb=cntb):
            _ffn_subblock(x_ref, wib, wob, acc_ref, y_ref, f, rows, s0, cntb)

        @pl.when(jnp.logical_not(active) & (f == NF - 1))
        def _(rows=rows):
            y_ref[rows, :] = jnp.zeros((RSUB, D), jnp.float32)


def _ffn(counts, x, wi, wo):
    return pl.pallas_call(
        _ffn_body,
        grid=(E, NF),
        in_specs=[
            pl.BlockSpec(memory_space=pltpu.SMEM),
            pl.BlockSpec((B * CAP, D), lambda e, f: (e, 0)),
            pl.BlockSpec((1, D, FB), lambda e, f: (e, 0, f)),
            pl.BlockSpec((1, FB, D), lambda e, f: (e, f, 0)),
        ],
        out_specs=pl.BlockSpec((B * CAP, D), lambda e, f: (e, 0)),
        out_shape=jax.ShapeDtypeStruct((SLOTS, D), jnp.float32),
        scratch_shapes=[pltpu.VMEM((B * CAP, D), jnp.float32)],
        compiler_params=pltpu.CompilerParams(
            dimension_semantics=("arbitrary", "arbitrary"),
        ),
    )(counts, x, wi, wo)


def _sc_combine(y, destc):
    mesh = plsc.VectorSubcoreMesh(core_axis_name="c", subcore_axis_name="s")

    @functools.partial(
        pl.kernel, mesh=mesh,
        out_type=jax.ShapeDtypeStruct((BS, D), jnp.float32),
        scratch_types=[
            pltpu.VMEM((NCH, CHUNK), jnp.int32),
            pltpu.VMEM((CHUNK, D), jnp.float32),
            pltpu.SemaphoreType.DMA,
        ],
    )
    def comb(y_hbm, idx_hbm, g_hbm, idx_v, rows_v, sem):
        wid = lax.axis_index("s") * NC + lax.axis_index("c")
        base = wid * TPW
        for j in range(NCH):
            off = base + j * CHUNK
            pltpu.sync_copy(idx_hbm.at[pl.ds(off, CHUNK)], idx_v.at[j])
            pltpu.async_copy(y_hbm.at[idx_v.at[j]], rows_v, sem).wait()
            pltpu.sync_copy(rows_v, g_hbm.at[pl.ds(off, CHUNK), :])

    return comb(y, destc)


def kernel(hidden_states, Wc, wi, wo):
    h2 = hidden_states.reshape(BS, D)
    wcp = jnp.pad(Wc, ((0, 0), (0, LANES - E)))
    logits, hs, dest, destc, counts = _router(h2, wcp)
    x = _sc_dispatch(hs, dest.reshape(BS))
    y = _ffn(counts, x, wi, wo)
    out = _sc_combine(y, destc.reshape(BS))
    return out.reshape(B, S, D), logits[:, :E].reshape(B, S, E)


# speedup vs baseline: 1.0204x; 1.0204x over previous
import functools

import jax
import jax.numpy as jnp
from jax import lax
from jax.experimental import pallas as pl
from jax.experimental.pallas import tpu as pltpu
from jax.experimental.pallas import tpu_sc as plsc

B, S, D, F, E = 2, 2048, 1024, 4096, 8
CAP = 512
BS = B * S
LANES = 128
SLOTS = E * B * CAP
TRASH = SLOTS
XROWS = SLOTS + 1024
SCHUNK = 512
NC, NS = 2, 16
NW = NC * NS
TPW = BS // NW
CHUNK = 64
NCH = TPW // CHUNK
FB = 1024
NF = F // FB
RSUB = 256


def _router_body(h_ref, wc_ref, logits_ref, hs_ref, dest_ref, destc_ref,
                 counts_ref):
    h = h_ref[...]
    logits = jnp.dot(h, wc_ref[...], preferred_element_type=jnp.float32)
    logits_ref[...] = logits
    lane = lax.broadcasted_iota(jnp.int32, (BS, LANES), 1)
    ml = jnp.where(lane < E, logits, -jnp.inf)
    m = jnp.max(ml, axis=1, keepdims=True)
    eidx = jnp.min(jnp.where(ml == m, lane, LANES), axis=1, keepdims=True)
    pmax = 1.0 / jnp.sum(jnp.exp(ml - m), axis=1, keepdims=True)
    oh = (lane == eidx).astype(jnp.float32)
    ri = lax.broadcasted_iota(jnp.int32, (SCHUNK, SCHUNK), 0)
    ci = lax.broadcasted_iota(jnp.int32, (SCHUNK, SCHUNK), 1)
    tri = (ri >= ci).astype(jnp.float32)
    counts_ref[...] = jnp.zeros((8, LANES), jnp.int32)
    for b in range(B):
        carry = jnp.zeros((1, LANES), jnp.float32)
        for c in range(S // SCHUNK):
            r0 = b * S + c * SCHUNK
            seg = oh[r0:r0 + SCHUNK, :]
            p = jnp.dot(tri, seg, preferred_element_type=jnp.float32) + carry
            carry = carry + jnp.sum(seg, axis=0, keepdims=True)
            prio = jnp.sum(p * seg, axis=1, keepdims=True)
            kept = prio <= CAP
            slot = prio.astype(jnp.int32) - 1
            es = eidx[r0:r0 + SCHUNK, :]
            dst = es * (B * CAP) + b * CAP + slot
            dest_ref[r0:r0 + SCHUNK, :] = jnp.where(kept, dst, TRASH)
            hscale = jnp.where(kept, pmax[r0:r0 + SCHUNK, :], 0.0)
            hs_ref[r0:r0 + SCHUNK, :] = h[r0:r0 + SCHUNK, :] * hscale
        cnt = jnp.minimum(carry, float(CAP)).astype(jnp.int32)
        counts_ref[b:b + 1, :] = cnt
        zlane = jnp.min(jnp.where((carry < CAP) & (lane[:1, :] < E),
                                  lane[:1, :], LANES), axis=1, keepdims=True)
        zcnt = jnp.sum(jnp.where(lane[:1, :] == zlane, carry, 0.0),
                       axis=1, keepdims=True).astype(jnp.int32)
        zrow = zlane * (B * CAP) + b * CAP + zcnt
        for c in range(S // SCHUNK):
            r0 = b * S + c * SCHUNK
            dv = dest_ref[r0:r0 + SCHUNK, :]
            destc_ref[r0:r0 + SCHUNK, :] = jnp.where(dv == TRASH, zrow, dv)


def _router(h2, wcp):
    return pl.pallas_call(
        _router_body,
        out_shape=[
            jax.ShapeDtypeStruct((BS, LANES), jnp.float32),
            jax.ShapeDtypeStruct((BS, D), jnp.float32),
            jax.ShapeDtypeStruct((BS, 1), jnp.int32),
            jax.ShapeDtypeStruct((BS, 1), jnp.int32),
            jax.ShapeDtypeStruct((8, LANES), jnp.int32),
        ],
    )(h2, wcp)


def _sc_dispatch(h2, dest):
    mesh = plsc.VectorSubcoreMesh(core_axis_name="c", subcore_axis_name="s")

    @functools.partial(
        pl.kernel, mesh=mesh,
        out_type=jax.ShapeDtypeStruct((XROWS, D), jnp.float32),
        scratch_types=[
            pltpu.VMEM((NCH, CHUNK), jnp.int32),
            pltpu.VMEM((CHUNK, D), jnp.float32),
            pltpu.SemaphoreType.DMA,
        ],
    )
    def disp(h_hbm, dest_hbm, x_hbm, idx_v, rows_v, sem):
        wid = lax.axis_index("s") * NC + lax.axis_index("c")
        base = wid * TPW
        for j in range(NCH):
            off = base + j * CHUNK
            pltpu.sync_copy(dest_hbm.at[pl.ds(off, CHUNK)], idx_v.at[j])
            pltpu.sync_copy(h_hbm.at[pl.ds(off, CHUNK), :], rows_v)
            pltpu.async_copy(rows_v, x_hbm.at[idx_v.at[j]], sem).wait()

    return disp(h2, dest)


def _ffn_subblock(x_ref, wib, wob, acc_ref, y_ref, f, rows, s0, cntb):
    slot = s0 + lax.broadcasted_iota(jnp.int32, (RSUB, 1), 0)
    xm = jnp.where(slot < cntb, x_ref[rows, :], 0.0).astype(jnp.bfloat16)
    hmid = jnp.maximum(
        jnp.dot(xm, wib, preferred_element_type=jnp.float32), 0.0)
    out = jnp.dot(hmid.astype(jnp.bfloat16), wob,
                  preferred_element_type=jnp.float32)

    @pl.when(f == 0)
    def _():
        acc_ref[rows, :] = out

    @pl.when((f > 0) & (f < NF - 1))
    def _():
        acc_ref[rows, :] = acc_ref[rows, :] + out

    @pl.when(f == NF - 1)
    def _():
        y_ref[rows, :] = acc_ref[rows, :] + out


def _ffn_body(counts_ref, x_ref, wi_ref, wo_ref, y_ref, acc_ref):
    e = pl.program_id(0)
    f = pl.program_id(1)
    cnt = [counts_ref[0, e], counts_ref[1, e]]
    wib = wi_ref[0].astype(jnp.bfloat16)
    wob = wo_ref[0].astype(jnp.bfloat16)
    for sb in range(B * CAP // RSUB):
        b, s0 = (sb * RSUB) // CAP, (sb * RSUB) % CAP
        rows = pl.ds(sb * RSUB, RSUB)
        cntb = cnt[b]
        active = cntb > s0

        @pl.when(active)
        def _(rows=rows, s0=s0, cntb=cntb):
            _ffn_subblock(x_ref, wib, wob, acc_ref, y_ref, f, rows, s0, cntb)

        @pl.when(jnp.logical_not(active) & (f == NF - 1))
        def _(rows=rows):
            y_ref[rows, :] = jnp.zeros((RSUB, D), jnp.float32)


def _ffn(counts, x, wi, wo):
    return pl.pallas_call(
        _ffn_body,
        grid=(E, NF),
        in_specs=[
            pl.BlockSpec(memory_space=pltpu.SMEM),
            pl.BlockSpec((B * CAP, D), lambda e, f: (e, 0)),
            pl.BlockSpec((1, D, FB), lambda e, f: (e, 0, f)),
            pl.BlockSpec((1, FB, D), lambda e, f: (e, f, 0)),
        ],
        out_specs=pl.BlockSpec((B * CAP, D), lambda e, f: (e, 0)),
        out_shape=jax.ShapeDtypeStruct((SLOTS, D), jnp.float32),
        scratch_shapes=[pltpu.VMEM((B * CAP, D), jnp.float32)],
        compiler_params=pltpu.CompilerParams(
            dimension_semantics=("arbitrary", "arbitrary"),
        ),
    )(counts, x, wi, wo)


def _sc_combine(y, destc):
    mesh = plsc.VectorSubcoreMesh(core_axis_name="c", subcore_axis_name="s")

    @functools.partial(
        pl.kernel, mesh=mesh,
        out_type=jax.ShapeDtypeStruct((BS, D), jnp.float32),
        scratch_types=[
            pltpu.VMEM((NCH, CHUNK), jnp.int32),
            pltpu.VMEM((CHUNK, D), jnp.float32),
            pltpu.SemaphoreType.DMA,
        ],
    )
    def comb(y_hbm, idx_hbm, g_hbm, idx_v, rows_v, sem):
        wid = lax.axis_index("s") * NC + lax.axis_index("c")
        base = wid * TPW
        for j in range(NCH):
            off = base + j * CHUNK
            pltpu.sync_copy(idx_hbm.at[pl.ds(off, CHUNK)], idx_v.at[j])
            pltpu.async_copy(y_hbm.at[idx_v.at[j]], rows_v, sem).wait()
            pltpu.sync_copy(rows_v, g_hbm.at[pl.ds(off, CHUNK), :])

    return comb(y, destc)


def kernel(hidden_states, Wc, wi, wo):
    h2 = hidden_states.reshape(BS, D)
    wcp = jnp.pad(Wc, ((0, 0), (0, LANES - E)))
    logits, hs, dest, destc, counts = _router(h2, wcp)
    x = _sc_dispatch(hs, dest.reshape(BS))
    y = _ffn(counts, x, wi, wo)
    out = _sc_combine(y, destc.reshape(BS))
    return out.reshape(B, S, D), logits[:, :E].reshape(B, S, E)
